# Initial kernel scaffold; baseline (speedup 1.0000x reference)
import jax
import jax.numpy as jnp
from jax.experimental import pallas as pl


def kernel(x, table, W, b):
    raise NotImplementedError("write your pallas kernel here")


# speedup vs baseline: 12.5153x; 12.5153x over previous
import functools

import jax
import jax.numpy as jnp
from jax import lax
from jax.experimental import pallas as pl
from jax.experimental.pallas import tpu as pltpu
from jax.experimental.pallas import tpu_sc as plsc

VOCAB = 1000000
EMBED_DIM = 32
NUM_LABELS = 10
BATCH = 16384
HIST = 50
BL = BATCH * HIST

PROJ_DIM = 16
_ROWS_PER_BLOCK = 8000


def _proj_body(t_ref, w_ref, b_ref, o_ref):
    o_ref[...] = (
        jnp.dot(t_ref[...], w_ref[...], preferred_element_type=jnp.float32)
        + b_ref[...]
    )


def _project_table(table, W, b):
    wp = jnp.zeros((EMBED_DIM, PROJ_DIM), jnp.float32).at[:, :NUM_LABELS].set(W)
    bp = jnp.zeros((1, PROJ_DIM), jnp.float32).at[:, :NUM_LABELS].set(b)
    return pl.pallas_call(
        _proj_body,
        grid=(VOCAB // _ROWS_PER_BLOCK,),
        in_specs=[
            pl.BlockSpec((_ROWS_PER_BLOCK, EMBED_DIM), lambda i: (i, 0)),
            pl.BlockSpec((EMBED_DIM, PROJ_DIM), lambda i: (0, 0)),
            pl.BlockSpec((1, PROJ_DIM), lambda i: (0, 0)),
        ],
        out_specs=pl.BlockSpec((_ROWS_PER_BLOCK, PROJ_DIM), lambda i: (i, 0)),
        out_shape=jax.ShapeDtypeStruct((VOCAB, PROJ_DIM), jnp.float32),
    )(table, wp, bp)


_NC, _NS = 2, 16
_NW = _NC * _NS
_B_PER_W = BL // _NW
_CHUNK = 3200
_NCHUNK = _B_PER_W // _CHUNK


def _gather_kernel(idx_hbm, p_hbm, out_hbm, idx_v, rows_v, sem):
    wid = lax.axis_index("s") * _NC + lax.axis_index("c")
    base = wid * _B_PER_W

    def body(i, carry):
        off = base + i * _CHUNK
        pltpu.sync_copy(idx_hbm.at[pl.ds(off, _CHUNK)], idx_v)
        pltpu.async_copy(p_hbm.at[idx_v], rows_v, sem).wait()
        pltpu.sync_copy(rows_v, out_hbm.at[pl.ds(off, _CHUNK)])
        return carry

    lax.fori_loop(0, _NCHUNK, body, 0)


def _gather_rows(idx_flat, p):
    mesh = plsc.VectorSubcoreMesh(core_axis_name="c", subcore_axis_name="s")
    fn = functools.partial(
        pl.kernel,
        mesh=mesh,
        out_type=jax.ShapeDtypeStruct((BL, PROJ_DIM), jnp.float32),
        scratch_types=[
            pltpu.VMEM((_CHUNK,), jnp.int32),
            pltpu.VMEM((_CHUNK, PROJ_DIM), jnp.float32),
            pltpu.SemaphoreType.DMA,
        ],
        compiler_params=pltpu.CompilerParams(use_tc_tiling_on_sc=False),
    )(_gather_kernel)
    return fn(idx_flat, p)


def kernel(x, table, W, b):
    p = _project_table(table, W, b)
    out = _gather_rows(x.reshape(BL), p)
    return out[:, :NUM_LABELS].reshape(BATCH, HIST, NUM_LABELS)


# speedup vs baseline: 14.3526x; 1.1468x over previous
import functools

import jax
import jax.numpy as jnp
from jax import lax
from jax.experimental import pallas as pl
from jax.experimental.pallas import tpu as pltpu
from jax.experimental.pallas import tpu_sc as plsc

VOCAB = 1000000
EMBED_DIM = 32
NUM_LABELS = 10
BATCH = 16384
HIST = 50
BL = BATCH * HIST

PROJ_DIM = 16
_RPB = 8000
_SUB = _RPB // 8


def _proj_body(t_ref, w_ref, b_ref, o_ref):
    for j in range(8):
        d = (
            jnp.dot(
                t_ref[pl.ds(j * _SUB, _SUB), :],
                w_ref[...],
                preferred_element_type=jnp.float32,
            )
            + b_ref[...]
        )
        o_ref[:, j * PROJ_DIM : (j + 1) * PROJ_DIM] = d


def _project_table(table, W, b):
    wp = jnp.zeros((EMBED_DIM, PROJ_DIM), jnp.float32).at[:, :NUM_LABELS].set(W)
    bp = jnp.zeros((1, PROJ_DIM), jnp.float32).at[:, :NUM_LABELS].set(b)
    p2 = pl.pallas_call(
        _proj_body,
        grid=(VOCAB // _RPB,),
        in_specs=[
            pl.BlockSpec((_RPB, EMBED_DIM), lambda i: (i, 0)),
            pl.BlockSpec((EMBED_DIM, PROJ_DIM), lambda i: (0, 0)),
            pl.BlockSpec((1, PROJ_DIM), lambda i: (0, 0)),
        ],
        out_specs=pl.BlockSpec((_SUB, 128), lambda i: (i, 0)),
        out_shape=jax.ShapeDtypeStruct((VOCAB // 8, 128), jnp.float32),
    )(table, wp, bp)
    return p2.reshape(VOCAB, PROJ_DIM)


_NC, _NS = 2, 16
_NW = _NC * _NS
_B_PER_W = BL // _NW
_CHUNK = 3200
_NCHUNK = _B_PER_W // _CHUNK
_PK = _CHUNK * NUM_LABELS
_GROUPS = _CHUNK // 8
_NVEC = _CHUNK // 16


def _gather_kernel(idx_hbm, p_hbm, out_hbm, idx_v, idx2_v, rows_v, pk_v, sem):
    wid = lax.axis_index("s") * _NC + lax.axis_index("c")
    base = wid * _B_PER_W

    lane = jnp.arange(16, dtype=jnp.int32)
    rowbases = []
    colbases = []
    for k in range(5):
        j = lane + (16 * k)
        row = (j * 205) >> 11
        rowbases.append(row)
        colbases.append(j - row * 10)

    def chunk_body(i, carry):
        off = base + i * _CHUNK
        pltpu.sync_copy(idx_hbm.at[pl.ds(off, _CHUNK)], idx_v)

        def perm_body(t, c2):
            v = idx_v[pl.ds(t * 16, 16)]
            bi = v // 8000
            rem = v - bi * 8000
            bj = rem // 1000
            br = rem - bj * 1000
            idx2_v[pl.ds(t * 16, 16)] = bi * 8000 + br * 8 + bj
            return c2

        lax.fori_loop(0, _NVEC, perm_body, 0)
        pltpu.async_copy(p_hbm.at[idx2_v], rows_v, sem).wait()

        def pack_body(g, c2):
            r0 = g * 8
            w0 = g * 80
            for k in range(5):
                v = plsc.load_gather(rows_v, [r0 + rowbases[k], colbases[k]])
                pk_v[pl.ds(w0 + 16 * k, 16)] = v
            return c2

        lax.fori_loop(0, _GROUPS, pack_body, 0)
        pltpu.sync_copy(pk_v, out_hbm.at[pl.ds(off * NUM_LABELS, _PK)])
        return carry

    lax.fori_loop(0, _NCHUNK, chunk_body, 0)


def _gather_rows(idx_flat, p):
    mesh = plsc.VectorSubcoreMesh(core_axis_name="c", subcore_axis_name="s")
    fn = functools.partial(
        pl.kernel,
        mesh=mesh,
        out_type=jax.ShapeDtypeStruct((BL * NUM_LABELS,), jnp.float32),
        scratch_types=[
            pltpu.VMEM((_CHUNK,), jnp.int32),
            pltpu.VMEM((_CHUNK,), jnp.int32),
            pltpu.VMEM((_CHUNK, PROJ_DIM), jnp.float32),
            pltpu.VMEM((_PK,), jnp.float32),
            pltpu.SemaphoreType.DMA,
        ],
        compiler_params=pltpu.CompilerParams(
            use_tc_tiling_on_sc=False, needs_layout_passes=False
        ),
    )(_gather_kernel)
    return fn(idx_flat, p)


def kernel(x, table, W, b):
    p = _project_table(table, W, b)
    out = _gather_rows(x.reshape(BL), p)
    return out.reshape(BATCH, HIST, NUM_LABELS)


# speedup vs baseline: 38.9853x; 2.7163x over previous
import functools

import jax
import jax.numpy as jnp
from jax import lax
from jax.experimental import pallas as pl
from jax.experimental.pallas import tpu as pltpu
from jax.experimental.pallas import tpu_sc as plsc

VOCAB = 1000000
EMBED_DIM = 32
NUM_LABELS = 10
BATCH = 16384
HIST = 50
BL = BATCH * HIST

PROJ_DIM = 16
_VPB = 8192
_SUB = _VPB // 8
_NBLK = -(-VOCAB // _VPB)
_PROWS = _NBLK * _SUB


def _proj_body(t_ref, w_ref, b_ref, o_ref):
    for j in range(8):
        d = lax.dot_general(
            t_ref[:, pl.ds(j * _SUB, _SUB)],
            w_ref[...],
            (((0,), (0,)), ((), ())),
            preferred_element_type=jnp.float32,
        )
        o_ref[:, j * PROJ_DIM : (j + 1) * PROJ_DIM] = d + b_ref[...]


def _project_table(tableT, W, b):
    wp = jnp.zeros((EMBED_DIM, PROJ_DIM), jnp.float32).at[:, :NUM_LABELS].set(W)
    bp = jnp.zeros((1, PROJ_DIM), jnp.float32).at[:, :NUM_LABELS].set(b)
    p2 = pl.pallas_call(
        _proj_body,
        grid=(_NBLK,),
        in_specs=[
            pl.BlockSpec((EMBED_DIM, _VPB), lambda i: (0, i)),
            pl.BlockSpec((EMBED_DIM, PROJ_DIM), lambda i: (0, 0)),
            pl.BlockSpec((1, PROJ_DIM), lambda i: (0, 0)),
        ],
        out_specs=pl.BlockSpec((_SUB, 128), lambda i: (i, 0)),
        out_shape=jax.ShapeDtypeStruct((_PROWS, 128), jnp.float32),
    )(tableT, wp, bp)
    return p2.reshape(_PROWS * 8, PROJ_DIM)


_NC, _NS = 2, 16
_NW = _NC * _NS
_BW = BATCH // _NW


def _gather_kernel(idx_hbm, p_hbm, out_hbm, idx_v, rows_v, ob_v, sem):
    wid = lax.axis_index("s") * _NC + lax.axis_index("c")
    b0 = wid * _BW

    lane = jnp.arange(16, dtype=jnp.int32)

    def l_body(l, carry):
        pltpu.sync_copy(idx_hbm.at[pl.ds(l * BATCH + b0, _BW)], idx_v)
        pltpu.async_copy(p_hbm.at[idx_v], rows_v, sem).wait()

        def pack_body(kk, c2):
            row = kk * 16 + lane
            for c in range(NUM_LABELS):
                v = plsc.load_gather(rows_v, [row, jnp.full((16,), c, jnp.int32)])
                ob_v[c, pl.ds(kk * 16, 16)] = v
            return c2

        lax.fori_loop(0, _BW // 16, pack_body, 0)
        pltpu.sync_copy(ob_v, out_hbm.at[:, pl.ds(l * BATCH + b0, _BW)])
        return carry

    lax.fori_loop(0, HIST, l_body, 0)


def _gather_rows(idx_flat, p):
    mesh = plsc.VectorSubcoreMesh(core_axis_name="c", subcore_axis_name="s")
    fn = functools.partial(
        pl.kernel,
        mesh=mesh,
        out_type=jax.ShapeDtypeStruct((NUM_LABELS, BL), jnp.float32),
        scratch_types=[
            pltpu.VMEM((_BW,), jnp.int32),
            pltpu.VMEM((_BW, PROJ_DIM), jnp.float32),
            pltpu.VMEM((NUM_LABELS, _BW), jnp.float32),
            pltpu.SemaphoreType.DMA,
        ],
        compiler_params=pltpu.CompilerParams(
            use_tc_tiling_on_sc=False, needs_layout_passes=False
        ),
    )(_gather_kernel)
    return fn(idx_flat, p)


def kernel(x, table, W, b):
    p = _project_table(table.T, W, b)
    v = x.T.reshape(BL)
    xp = (v & ~jnp.int32(8191)) + ((v & 1023) << 3) + ((v >> 10) & 7)
    out = _gather_rows(xp, p)
    return out.reshape(NUM_LABELS, HIST, BATCH).transpose(2, 1, 0)


# speedup vs baseline: 44.4408x; 1.1399x over previous
import functools

import jax
import jax.numpy as jnp
from jax import lax
from jax.experimental import pallas as pl
from jax.experimental.pallas import tpu as pltpu
from jax.experimental.pallas import tpu_sc as plsc

VOCAB = 1000000
EMBED_DIM = 32
NUM_LABELS = 10
BATCH = 16384
HIST = 50
BL = BATCH * HIST

PROJ_DIM = 16
_VPB = 16384
_SUB = _VPB // 8
_SUBSH = 11
_NBLK = -(-VOCAB // _VPB)
_PROWS = _NBLK * _SUB


def _proj_body(t_ref, w_ref, b_ref, o_ref):
    for j in range(8):
        d = lax.dot_general(
            t_ref[:, pl.ds(j * _SUB, _SUB)],
            w_ref[...],
            (((0,), (0,)), ((), ())),
            preferred_element_type=jnp.float32,
        )
        o_ref[:, j * PROJ_DIM : (j + 1) * PROJ_DIM] = d + b_ref[...]


def _project_table(tableT, W, b):
    wp = jnp.zeros((EMBED_DIM, PROJ_DIM), jnp.float32).at[:, :NUM_LABELS].set(W)
    bp = jnp.zeros((1, PROJ_DIM), jnp.float32).at[:, :NUM_LABELS].set(b)
    p2 = pl.pallas_call(
        _proj_body,
        grid=(_NBLK,),
        in_specs=[
            pl.BlockSpec((EMBED_DIM, _VPB), lambda i: (0, i)),
            pl.BlockSpec((EMBED_DIM, PROJ_DIM), lambda i: (0, 0)),
            pl.BlockSpec((1, PROJ_DIM), lambda i: (0, 0)),
        ],
        out_specs=pl.BlockSpec((_SUB, 128), lambda i: (i, 0)),
        out_shape=jax.ShapeDtypeStruct((_PROWS, 128), jnp.float32),
    )(tableT, wp, bp)
    return p2.reshape(_PROWS * 8, PROJ_DIM)


_NC, _NS = 2, 16
_NW = _NC * _NS
_BW = BATCH // _NW


def _gather_kernel(
    idx_hbm, p_hbm, out_hbm,
    idx0_v, idx1_v, rows0_v, rows1_v, ob0_v, ob1_v, sem0, sem1,
):
    wid = lax.axis_index("s") * _NC + lax.axis_index("c")
    b0 = wid * _BW

    lane = jnp.arange(16, dtype=jnp.int32)
    cvecs = [jnp.full((16,), c, jnp.int32) for c in range(NUM_LABELS)]

    def pack_and_emit(rows_v, ob_v, l):
        def pack_body(kk, c2):
            row = kk * 16 + lane
            for c in range(NUM_LABELS):
                ob_v[c, pl.ds(kk * 16, 16)] = plsc.load_gather(
                    rows_v, [row, cvecs[c]]
                )
            return c2

        lax.fori_loop(0, _BW // 16, pack_body, 0)
        pltpu.sync_copy(ob_v, out_hbm.at[:, pl.ds(l * BATCH + b0, _BW)])

    pltpu.sync_copy(idx_hbm.at[pl.ds(b0, _BW)], idx0_v)
    pltpu.async_copy(p_hbm.at[idx0_v], rows0_v, sem0)

    def pair_body(l2, carry):
        l0 = l2 * 2
        l1 = l0 + 1
        pltpu.sync_copy(idx_hbm.at[pl.ds(l1 * BATCH + b0, _BW)], idx1_v)
        pltpu.async_copy(p_hbm.at[idx1_v], rows1_v, sem1)
        pltpu.make_async_copy(p_hbm.at[idx0_v], rows0_v, sem0).wait()
        pack_and_emit(rows0_v, ob0_v, l0)

        @pl.when(l2 < HIST // 2 - 1)
        def _():
            pltpu.sync_copy(idx_hbm.at[pl.ds((l0 + 2) * BATCH + b0, _BW)], idx0_v)
            pltpu.async_copy(p_hbm.at[idx0_v], rows0_v, sem0)

        pltpu.make_async_copy(p_hbm.at[idx1_v], rows1_v, sem1).wait()
        pack_and_emit(rows1_v, ob1_v, l1)
        return carry

    lax.fori_loop(0, HIST // 2, pair_body, 0)


def _gather_rows(idx_flat, p):
    mesh = plsc.VectorSubcoreMesh(core_axis_name="c", subcore_axis_name="s")
    fn = functools.partial(
        pl.kernel,
        mesh=mesh,
        out_type=jax.ShapeDtypeStruct((NUM_LABELS, BL), jnp.float32),
        scratch_types=[
            pltpu.VMEM((_BW,), jnp.int32),
            pltpu.VMEM((_BW,), jnp.int32),
            pltpu.VMEM((_BW, PROJ_DIM), jnp.float32),
            pltpu.VMEM((_BW, PROJ_DIM), jnp.float32),
            pltpu.VMEM((NUM_LABELS, _BW), jnp.float32),
            pltpu.VMEM((NUM_LABELS, _BW), jnp.float32),
            pltpu.SemaphoreType.DMA,
            pltpu.SemaphoreType.DMA,
        ],
        compiler_params=pltpu.CompilerParams(
            use_tc_tiling_on_sc=False, needs_layout_passes=False
        ),
    )(_gather_kernel)
    return fn(idx_flat, p)


def kernel(x, table, W, b):
    p = _project_table(table.T, W, b)
    v = x.T.reshape(BL)
    xp = (
        (v & ~jnp.int32(_VPB - 1))
        + ((v & (_SUB - 1)) << 3)
        + ((v >> _SUBSH) & 7)
    )
    out = _gather_rows(xp, p)
    return out.reshape(NUM_LABELS, HIST, BATCH).transpose(2, 1, 0)


# speedup vs baseline: 44.4551x; 1.0003x over previous
import functools

import jax
import jax.numpy as jnp
from jax import lax
from jax.experimental import pallas as pl
from jax.experimental.pallas import tpu as pltpu
from jax.experimental.pallas import tpu_sc as plsc

VOCAB = 1000000
EMBED_DIM = 32
NUM_LABELS = 10
BATCH = 16384
HIST = 50
BL = BATCH * HIST

PROJ_DIM = 16
_VPB = 16384
_SUB = _VPB // 8
_SUBSH = 11
_NBLK = -(-VOCAB // _VPB)
_PROWS = _NBLK * _SUB


def _proj_body(t_ref, w_ref, b_ref, o_ref):
    for j in range(8):
        d = lax.dot_general(
            t_ref[:, pl.ds(j * _SUB, _SUB)],
            w_ref[...],
            (((0,), (0,)), ((), ())),
            preferred_element_type=jnp.float32,
        )
        o_ref[:, j * PROJ_DIM : (j + 1) * PROJ_DIM] = d + b_ref[...]


def _project_table(tableT, W, b):
    wp = jnp.zeros((EMBED_DIM, PROJ_DIM), jnp.float32).at[:, :NUM_LABELS].set(W)
    bp = jnp.zeros((1, PROJ_DIM), jnp.float32).at[:, :NUM_LABELS].set(b)
    p2 = pl.pallas_call(
        _proj_body,
        grid=(_NBLK,),
        in_specs=[
            pl.BlockSpec((EMBED_DIM, _VPB), lambda i: (0, i)),
            pl.BlockSpec((EMBED_DIM, PROJ_DIM), lambda i: (0, 0)),
            pl.BlockSpec((1, PROJ_DIM), lambda i: (0, 0)),
        ],
        out_specs=pl.BlockSpec((_SUB, 128), lambda i: (i, 0)),
        out_shape=jax.ShapeDtypeStruct((_PROWS, 128), jnp.float32),
        compiler_params=pltpu.CompilerParams(fuse_transposed_lhs_in_matmul=True),
    )(tableT, wp, bp)
    return p2.reshape(_PROWS * 8, PROJ_DIM)


_NC, _NS = 2, 16
_NW = _NC * _NS
_BW = BATCH // _NW


_DEPTH = 5


def _gather_kernel(idx_hbm, p_hbm, out_hbm, idx_v, rows_v, ob_v, *sems):
    wid = lax.axis_index("s") * _NC + lax.axis_index("c")
    b0 = wid * _BW

    lane = jnp.arange(16, dtype=jnp.int32)
    cvecs = [jnp.full((16,), c, jnp.int32) for c in range(NUM_LABELS)]
    kvecs = [jnp.full((16,), k, jnp.int32) for k in range(_DEPTH)]

    def issue(k, l):
        pltpu.sync_copy(idx_hbm.at[pl.ds(l * BATCH + b0, _BW)], idx_v.at[k])
        pltpu.async_copy(p_hbm.at[idx_v.at[k]], rows_v.at[k], sems[k])

    def pack_and_emit(k, l):
        pltpu.make_async_copy(p_hbm.at[idx_v.at[k]], rows_v.at[k], sems[k]).wait()

        def pack_body(kk, c2):
            row = kk * 16 + lane
            for c in range(NUM_LABELS):
                ob_v[k, c, pl.ds(kk * 16, 16)] = plsc.load_gather(
                    rows_v, [kvecs[k], row, cvecs[c]]
                )
            return c2

        lax.fori_loop(0, _BW // 16, pack_body, 0)
        pltpu.sync_copy(ob_v.at[k], out_hbm.at[:, pl.ds(l * BATCH + b0, _BW)])

    for k in range(_DEPTH):
        issue(k, k)

    def group_body(g, carry):
        l0 = g * _DEPTH
        for k in range(_DEPTH):
            pack_and_emit(k, l0 + k)

            @pl.when(g < HIST // _DEPTH - 1)
            def _():
                issue(k, l0 + k + _DEPTH)

        return carry

    lax.fori_loop(0, HIST // _DEPTH, group_body, 0)


def _gather_rows(idx_flat, p):
    mesh = plsc.VectorSubcoreMesh(core_axis_name="c", subcore_axis_name="s")
    fn = functools.partial(
        pl.kernel,
        mesh=mesh,
        out_type=jax.ShapeDtypeStruct((NUM_LABELS, BL), jnp.float32),
        scratch_types=[
            pltpu.VMEM((_DEPTH, _BW), jnp.int32),
            pltpu.VMEM((_DEPTH, _BW, PROJ_DIM), jnp.float32),
            pltpu.VMEM((_DEPTH, NUM_LABELS, _BW), jnp.float32),
        ]
        + [pltpu.SemaphoreType.DMA] * _DEPTH,
        compiler_params=pltpu.CompilerParams(
            use_tc_tiling_on_sc=False, needs_layout_passes=False
        ),
    )(_gather_kernel)
    return fn(idx_flat, p)


def kernel(x, table, W, b):
    p = _project_table(table.T, W, b)
    v = x.T.reshape(BL)
    xp = (
        (v & ~jnp.int32(_VPB - 1))
        + ((v & (_SUB - 1)) << 3)
        + ((v >> _SUBSH) & 7)
    )
    out = _gather_rows(xp, p)
    return out.reshape(NUM_LABELS, HIST, BATCH).transpose(2, 1, 0)
